# baseline (speedup 1.0000x reference)
import jax
import jax.numpy as jnp
from jax.experimental import pallas as pl

_HW = [[68, 120], [34, 60], [17, 30]]
_STRIDES = [8.0, 16.0, 32.0]
_A0 = _HW[0][0] * _HW[0][1]
_A1 = _A0 + _HW[1][0] * _HW[1][1]
_A = _A1 + _HW[2][0] * _HW[2][1]
_C = 16
_B = 64
_AP = 10752
_RPB = _AP * _C // 128
_R = _B * _RPB
_BB = 2


def _grid_xy(a_i32):
    a_i32 = jnp.minimum(a_i32, _A - 1)
    in0 = a_i32 < _A0
    in1 = a_i32 < _A1
    stride = jnp.where(in0, _STRIDES[0], jnp.where(in1, _STRIDES[1], _STRIDES[2]))
    start = jnp.where(in0, 0.0, jnp.where(in1, float(_A0), float(_A1)))
    width = jnp.where(in0, float(_HW[0][1]), jnp.where(in1, float(_HW[1][1]),
                                                       float(_HW[2][1])))
    rel = a_i32.astype(jnp.float32) - start
    gy = jnp.floor(rel / width)
    gx = rel - gy * width
    return gx, gy, stride


def _decode_kernel(x_ref, ga_ref, st_ref, o_ref, xs_ref, ys_ref, ss_ref):
    lane = jax.lax.broadcasted_iota(jnp.int32, (1, 128), 1)
    chan = lane & 15
    m2 = chan < 2
    m4 = chan < 4
    ga = ga_ref[...]
    st = st_ref[...]
    for b in range(_BB):
        x = x_ref[pl.ds(b * _RPB, _RPB), :]
        v = jnp.where(m2, x + ga, jnp.exp(x))
        o_ref[pl.ds(b * _RPB, _RPB), :] = jnp.where(m4, v * st, x)

    @pl.when(pl.program_id(0) == 0)
    def _():
        ja = jax.lax.broadcasted_iota(jnp.int32, (1, _A), 1)
        agx, agy, astride = _grid_xy(ja)
        xs_ref[...] = agx
        ys_ref[...] = agy
        ss_ref[...] = astride


def _tables():
    r = jax.lax.broadcasted_iota(jnp.int32, (_RPB, 128), 0)
    lane = jax.lax.broadcasted_iota(jnp.int32, (_RPB, 128), 1)
    a = r * 8 + (lane >> 4)
    chan = lane & 15
    gx, gy, stride = _grid_xy(a)
    ga = jnp.where(chan == 0, gx, jnp.where(chan == 1, gy, 0.0))
    return ga, stride


@jax.jit
def _decode(x):
    f32 = jnp.float32
    ga, st = _tables()
    one = jax.lax.optimization_barrier(jnp.float32(1.0))
    x2 = jnp.pad(x, ((0, 0), (0, _AP - _A), (0, 0))).reshape(_R, 128) * one
    aux = pl.BlockSpec((_RPB, 128), lambda i: (0, 0))
    out2, xs, ys, ss = pl.pallas_call(
        _decode_kernel,
        grid=(_B // _BB,),
        in_specs=[pl.BlockSpec((_BB * _RPB, 128), lambda i: (i, 0)), aux, aux],
        out_specs=[
            pl.BlockSpec((_BB * _RPB, 128), lambda i: (i, 0)),
            pl.BlockSpec((1, _A), lambda i: (0, 0)),
            pl.BlockSpec((1, _A), lambda i: (0, 0)),
            pl.BlockSpec((1, _A), lambda i: (0, 0)),
        ],
        out_shape=[
            jax.ShapeDtypeStruct((_R, 128), f32),
            jax.ShapeDtypeStruct((1, _A), f32),
            jax.ShapeDtypeStruct((1, _A), f32),
            jax.ShapeDtypeStruct((1, _A), f32),
        ],
    )(x2, ga, st)
    out = out2.reshape(_B, _AP, _C)[:, :_A, :] * one
    return out, xs, ys, ss


def kernel(outputs):
    return _decode(outputs)


# speedup vs baseline: 1.4774x; 1.4774x over previous
import numpy as np
import jax
import jax.numpy as jnp
from jax.experimental import pallas as pl

_HW = [[68, 120], [34, 60], [17, 30]]
_STRIDES = [8.0, 16.0, 32.0]
_A0 = _HW[0][0] * _HW[0][1]
_A1 = _A0 + _HW[1][0] * _HW[1][1]
_A = _A1 + _HW[2][0] * _HW[2][1]
_C = 16
_B = 64
_BBLK = 1


def _grid_xy_np(a):
    a = np.minimum(a, _A - 1)
    in0 = a < _A0
    in1 = a < _A1
    stride = np.where(in0, _STRIDES[0], np.where(in1, _STRIDES[1], _STRIDES[2]))
    start = np.where(in0, 0.0, np.where(in1, float(_A0), float(_A1)))
    width = np.where(in0, float(_HW[0][1]), np.where(in1, float(_HW[1][1]),
                                                     float(_HW[2][1])))
    rel = a.astype(np.float32) - start
    gy = np.floor(rel / width)
    gx = rel - gy * width
    return (gx.astype(np.float32), gy.astype(np.float32),
            stride.astype(np.float32))


def _tables_np():
    a = np.arange(_A)[:, None] * np.ones((1, _C), np.int32)
    chan = np.arange(_C)[None, :] * np.ones((_A, 1), np.int32)
    gx, gy, stride = _grid_xy_np(a)
    ga = np.where(chan == 0, gx, np.where(chan == 1, gy, 0.0)).astype(np.float32)
    return ga[None], stride[None].astype(np.float32)


_GA_NP, _ST_NP = _tables_np()
_GX_NP, _GY_NP, _SS_NP = _grid_xy_np(np.arange(_A)[None, :])


def _decode_kernel(x_ref, ga_ref, st_ref, o_ref):
    chan = jax.lax.broadcasted_iota(jnp.int32, (1, 1, _C), 2)
    m2 = chan < 2
    m4 = chan < 4
    x = x_ref[...]
    v = jnp.where(m2, x + ga_ref[...], jnp.exp(x))
    o_ref[...] = jnp.where(m4, v * st_ref[...], x)


@jax.jit
def _decode(x):
    f32 = jnp.float32
    ga = jnp.asarray(_GA_NP)
    st = jnp.asarray(_ST_NP)
    aux = pl.BlockSpec((1, _A, _C), lambda i: (0, 0, 0))
    out = pl.pallas_call(
        _decode_kernel,
        grid=(_B // _BBLK,),
        in_specs=[pl.BlockSpec((_BBLK, _A, _C), lambda i: (i, 0, 0)), aux, aux],
        out_specs=pl.BlockSpec((_BBLK, _A, _C), lambda i: (i, 0, 0)),
        out_shape=jax.ShapeDtypeStruct((_B, _A, _C), f32),
    )(x, ga, st)
    return out, jnp.asarray(_GX_NP), jnp.asarray(_GY_NP), jnp.asarray(_SS_NP)


def kernel(outputs):
    return _decode(outputs)


# speedup vs baseline: 3.4610x; 2.3426x over previous
import jax
import jax.numpy as jnp
from jax.experimental import pallas as pl
from jax.experimental.pallas import tpu as pltpu

_HW = [[68, 120], [34, 60], [17, 30]]
_STRIDES = [8.0, 16.0, 32.0]
_A0 = _HW[0][0] * _HW[0][1]
_A1 = _A0 + _HW[1][0] * _HW[1][1]
_A = _A1 + _HW[2][0] * _HW[2][1]
_C = 16
_K = _A * _C
_B = 64
_BBLK = 8


def _grid_xy(a_i32):
    in0 = a_i32 < _A0
    in1 = a_i32 < _A1
    stride = jnp.where(in0, _STRIDES[0], jnp.where(in1, _STRIDES[1], _STRIDES[2]))
    start = jnp.where(in0, 0.0, jnp.where(in1, float(_A0), float(_A1)))
    width = jnp.where(in0, float(_HW[0][1]), jnp.where(in1, float(_HW[1][1]),
                                                       float(_HW[2][1])))
    rel = a_i32.astype(jnp.float32) - start
    gy = jnp.floor(rel / width)
    gx = rel - gy * width
    return gx, gy, stride


def _decode_kernel(x_ref, chan_ref, gadd_ref, stride_ref, o_ref,
                   xs_ref, ys_ref, st_ref):
    chan = chan_ref[...]
    x = x_ref[...]
    decoded = jnp.where(chan < 2, x + gadd_ref[...], jnp.exp(x))
    o_ref[...] = jnp.where(chan < 4, decoded * stride_ref[...], x)

    @pl.when(pl.program_id(0) == 0)
    def _():
        ja = jax.lax.broadcasted_iota(jnp.int32, (1, _A), 1)
        agx, agy, astride = _grid_xy(ja)
        xs_ref[...] = agx
        ys_ref[...] = agy
        st_ref[...] = astride


def _col_consts():
    j = jax.lax.broadcasted_iota(jnp.int32, (1, _K), 1)
    a = j >> 4
    chan = j & 15
    gx, gy, stride = _grid_xy(a)
    gadd = jnp.where(chan == 0, gx, jnp.where(chan == 1, gy, 0.0))
    return chan, gadd, stride


@jax.jit
def _decode(x2):
    f32 = jnp.float32
    chan, gadd, stride = _col_consts()
    aux = pl.BlockSpec((1, _K), lambda i: (0, 0))
    out, xs, ys, st = pl.pallas_call(
        _decode_kernel,
        grid=(_B // _BBLK,),
        in_specs=[pl.BlockSpec((_BBLK, _K), lambda i: (i, 0)), aux, aux, aux],
        out_specs=[
            pl.BlockSpec((_BBLK, _K), lambda i: (i, 0)),
            pl.BlockSpec((1, _A), lambda i: (0, 0)),
            pl.BlockSpec((1, _A), lambda i: (0, 0)),
            pl.BlockSpec((1, _A), lambda i: (0, 0)),
        ],
        out_shape=[
            jax.ShapeDtypeStruct((_B, _K), f32),
            jax.ShapeDtypeStruct((1, _A), f32),
            jax.ShapeDtypeStruct((1, _A), f32),
            jax.ShapeDtypeStruct((1, _A), f32),
        ],
        compiler_params=pltpu.CompilerParams(
            allow_input_fusion=[True, True, True, True]),
    )(x2, chan, gadd, stride)
    return out, xs, ys, st


def kernel(outputs):
    x2 = outputs.reshape(_B, _K)
    out, xs, ys, st = _decode(x2)
    return out.reshape(_B, _A, _C), xs, ys, st
